# baseline (speedup 1.0000x reference)
import jax
import jax.numpy as jnp
from jax import lax
from jax.experimental import pallas as pl
from jax.experimental.pallas import tpu as pltpu
from jax.experimental.pallas import tpu_sc as plsc

L = 16
NW = 32
G = 26
D = 256
CH = 80
NG = CH // L
GG = CH * G // L
M_TOTAL = 100000
NCHUNK = M_TOTAL // CH


def _sc_body(g_hbm, p_hbm, rt_hbm, ct_hbm,
             pool_out, tag_out, reu_out, unc_out,
             pv, rtv, ctv, fm_v, um_v,
             g0, g1, go0, go1, r0, r1, u0, u1, t0, t1,
             *sems):
    G_V = [g0, g1]
    GO_V = [go0, go1]
    R_V = [r0, r1]
    U_V = [u0, u1]
    T_V = [t0, t1]
    SEM = [sems[0:5], sems[5:10]]

    wid = lax.axis_index("s") * 2 + lax.axis_index("c")

    pltpu.sync_copy(p_hbm, pv)
    pltpu.sync_copy(rt_hbm, rtv)
    pltpu.sync_copy(ct_hbm, ctv)

    P = [pv[pl.ds(i * L, L)] for i in range(22)]
    (w00, w01, w02, w03, w10, w11, w12, w13, w20, w21, w22, w23,
     fx, fy, cx0, cy0, n0, n1, n2, f0b, f1b, f2b) = P

    n_my = (NCHUNK - wid + NW - 1) // NW
    iota = lax.iota(jnp.int32, L)

    def base_of(k):
        return (wid + k * NW) * CH

    def gin_desc(k, b):
        return pltpu.make_async_copy(g_hbm.at[pl.ds(base_of(k), CH)],
                                     G_V[b], SEM[b][0])

    def out_descs(k, b):
        base = base_of(k)
        return [
            pltpu.make_async_copy(GO_V[b], pool_out.at[pl.ds(base, CH)],
                                  SEM[b][1]),
            pltpu.make_async_copy(R_V[b], reu_out.at[pl.ds(base, CH)],
                                  SEM[b][2]),
            pltpu.make_async_copy(U_V[b], unc_out.at[pl.ds(base, CH)],
                                  SEM[b][3]),
            pltpu.make_async_copy(T_V[b], tag_out.at[pl.ds(base, CH)],
                                  SEM[b][4]),
        ]

    def bf16_round(v):
        u = plsc.bitcast(v, jnp.int32)
        r = u + (jnp.int32(0x7FFF) + ((u >> 16) & 1))
        return plsc.bitcast(r & jnp.int32(-65536), jnp.float32)

    def col(c):
        return jnp.full((L,), c, jnp.int32)

    def compute(b):
        @plsc.parallel_loop(0, NG, unroll=2)
        def mask_body(j):
            rows = j * L + iota
            x = plsc.load_gather(G_V[b], [rows, col(0)])
            y = plsc.load_gather(G_V[b], [rows, col(1)])
            z = plsc.load_gather(G_V[b], [rows, col(2)])
            xb = bf16_round(x)
            yb = bf16_round(y)
            zb = bf16_round(z)
            cx = xb * w00 + yb * w01 + zb * w02 + w03
            cy = xb * w10 + yb * w11 + zb * w12 + w13
            cz_ = xb * w20 + yb * w21 + zb * w22 + w23
            mask1 = cz_ > 1e-6
            cz = jnp.maximum(cz_, 1e-6)
            pxf = fx * (cx / cz) + cx0
            pyf = fy * (cy / cz) + cy0
            mask2 = ((pxf >= 0.0) & (pxf < 640.0)
                     & (pyf >= 0.0) & (pyf < 480.0))
            in_vox = ((x > n0) & (x < f0b) & (y > n1) & (y < f1b)
                      & (z > n2) & (z < f2b))
            mask_det = mask1 & mask2 & in_vox
            g23 = plsc.load_gather(G_V[b], [rows, col(23)])
            g24 = plsc.load_gather(G_V[b], [rows, col(24)])
            one = jnp.full((L,), 1.0, jnp.float32)
            half = jnp.full((L,), 0.5, jnp.float32)
            zero = jnp.zeros((L,), jnp.float32)
            plsc.store_scatter(G_V[b], [rows, col(24)],
                               jnp.where(in_vox, one, g24))
            T_V[b][pl.ds(j * L, L)] = jnp.where(
                mask_det, jnp.where(g23 == 1.0, half, zero), one)
            fm_v[pl.ds(j * L, L)] = jnp.where(in_vox, one, zero)
            um_v[pl.ds(j * L, L)] = jnp.where(mask_det, zero, one)

        @plsc.parallel_loop(0, GG, unroll=4)
        def g_body(kk):
            off = kk * L
            rowv = rtv[pl.ds(off, L)]
            colv = ctv[pl.ds(off, L)]
            fmv = plsc.load_gather(fm_v, [rowv])
            umv = plsc.load_gather(um_v, [rowv])
            gv = plsc.load_gather(G_V[b], [rowv, colv])
            plsc.store_scatter(GO_V[b], [rowv, colv], gv)
            plsc.store_scatter(R_V[b], [rowv, colv], gv * fmv)
            plsc.store_scatter(U_V[b], [rowv, colv], gv * umv)

    gin_desc(0, 0).start()

    def body(i, _):
        for bslot in (0, 1):
            k = i * 2 + bslot

            @pl.when(k < n_my)
            def _(k=k, bslot=bslot):
                gin_desc(k, bslot).wait()

                @pl.when(k >= 2)
                def _():
                    for dsc in out_descs(k - 2, bslot):
                        dsc.wait()

                @pl.when(k + 1 < n_my)
                def _():
                    gin_desc(k + 1, 1 - bslot).start()

                compute(bslot)
                for dsc in out_descs(k, bslot):
                    dsc.start()

        return 0

    lax.fori_loop(0, (n_my + 1) // 2, body, 0)

    for bslot in (0, 1):
        for k_last in (n_my - 2, n_my - 1):
            @pl.when((k_last >= 0) & (k_last % 2 == bslot))
            def _(k_last=k_last, bslot=bslot):
                for dsc in out_descs(k_last, bslot):
                    dsc.wait()


def _tc_body(prm_ref, g_ref, f_ref, o_ref):
    g = g_ref[...]
    x = g[:, 0:1]
    y = g[:, 1:2]
    z = g[:, 2:3]
    in_vox = ((x > prm_ref[0]) & (x < prm_ref[3])
              & (y > prm_ref[1]) & (y < prm_ref[4])
              & (z > prm_ref[2]) & (z < prm_ref[5]))
    fm = jnp.where(in_vox, 1.0, 0.0).astype(jnp.float32)
    o_ref[...] = f_ref[...] * fm


def kernel(gaussian_pool, instance_feature_pool, world2cam, cam_k,
           vox_origin, scene_size, mlvl_img_feat, anchor):
    M = gaussian_pool.shape[0]
    batch_size = mlvl_img_feat.shape[0]
    eps = jnp.float32(0.001)
    near = vox_origin + eps
    far = vox_origin + scene_size - eps
    w_rounded = world2cam[:3, :].astype(jnp.bfloat16).astype(jnp.float32)
    params = jnp.concatenate([
        w_rounded.reshape(-1),
        jnp.stack([cam_k[0, 0], cam_k[1, 1], cam_k[0, 2], cam_k[1, 2]]),
        near, far,
    ]).astype(jnp.float32)
    params_b = jnp.repeat(params, L)

    rowtab = jnp.repeat(jnp.arange(CH, dtype=jnp.int32), G)
    coltab = jnp.tile(jnp.arange(G, dtype=jnp.int32), CH)

    mesh = plsc.VectorSubcoreMesh(core_axis_name="c", subcore_axis_name="s")
    f32 = jnp.float32
    i32 = jnp.int32
    dma_sems = [pltpu.SemaphoreType.DMA] * 10
    sc_call = pl.kernel(
        _sc_body,
        out_type=[
            jax.ShapeDtypeStruct((M, G), f32),
            jax.ShapeDtypeStruct((M,), f32),
            jax.ShapeDtypeStruct((M, G), f32),
            jax.ShapeDtypeStruct((M, G), f32),
        ],
        mesh=mesh,
        compiler_params=pltpu.CompilerParams(needs_layout_passes=False),
        scratch_types=[
            pltpu.VMEM((22 * L,), f32),
            pltpu.VMEM((CH * G,), i32),
            pltpu.VMEM((CH * G,), i32),
            pltpu.VMEM((CH,), f32),
            pltpu.VMEM((CH,), f32),
            pltpu.VMEM((CH, G), f32), pltpu.VMEM((CH, G), f32),
            pltpu.VMEM((CH, G), f32), pltpu.VMEM((CH, G), f32),
            pltpu.VMEM((CH, G), f32), pltpu.VMEM((CH, G), f32),
            pltpu.VMEM((CH, G), f32), pltpu.VMEM((CH, G), f32),
            pltpu.VMEM((CH,), f32), pltpu.VMEM((CH,), f32),
        ] + dma_sems,
    )
    pool_u, tag, reu, unc = sc_call(gaussian_pool, params_b, rowtab, coltab)

    TB = 400
    tc_params = jnp.concatenate([near, far]).astype(f32)
    inst = pl.pallas_call(
        _tc_body,
        out_shape=jax.ShapeDtypeStruct((M, D), f32),
        grid=(M // TB,),
        in_specs=[
            pl.BlockSpec(memory_space=pltpu.SMEM),
            pl.BlockSpec((TB, G), lambda i: (i, 0)),
            pl.BlockSpec((TB, D), lambda i: (i, 0)),
        ],
        out_specs=pl.BlockSpec((TB, D), lambda i: (i, 0)),
    )(tc_params, gaussian_pool, instance_feature_pool)

    anchor_tiled = jnp.tile(anchor[None], (batch_size, 1, 1))
    return (pool_u, tag, reu, unc, inst, anchor_tiled)


# speedup vs baseline: 1.3335x; 1.3335x over previous
import jax
import jax.numpy as jnp
from jax import lax
from jax.experimental import pallas as pl
from jax.experimental.pallas import tpu as pltpu
from jax.experimental.pallas import tpu_sc as plsc

L = 16
NW = 32
G = 26
D = 256
CH = 80
NG = CH // L
GG = CH * G // L
M_TOTAL = 100000
NCHUNK = M_TOTAL // CH


def _sc_body(g_hbm, f_hbm, p_hbm, rt_hbm, ct_hbm,
             pool_out, tag_out, reu_out, unc_out, inst_out,
             pv, rtv, ctv, fm_v, um_v,
             g0, g1, go0, go1, r0, r1, u0, u1, f0, f1, t0, t1,
             *sems):
    G_V = [g0, g1]
    GO_V = [go0, go1]
    R_V = [r0, r1]
    U_V = [u0, u1]
    F_V = [f0, f1]
    T_V = [t0, t1]
    SEM = [sems[0:7], sems[7:14]]

    wid = lax.axis_index("s") * 2 + lax.axis_index("c")

    pltpu.sync_copy(p_hbm, pv)
    pltpu.sync_copy(rt_hbm, rtv)
    pltpu.sync_copy(ct_hbm, ctv)

    P = [pv[pl.ds(i * L, L)] for i in range(22)]
    (w00, w01, w02, w03, w10, w11, w12, w13, w20, w21, w22, w23,
     fx, fy, cx0, cy0, n0, n1, n2, f0b, f1b, f2b) = P

    n_my = (NCHUNK - wid + NW - 1) // NW
    iota = lax.iota(jnp.int32, L)

    def base_of(k):
        return (wid + k * NW) * CH

    def gin_desc(k, b):
        return pltpu.make_async_copy(g_hbm.at[pl.ds(base_of(k), CH)],
                                     G_V[b], SEM[b][0])

    def fin_desc(k, b):
        return pltpu.make_async_copy(f_hbm.at[pl.ds(base_of(k), CH)],
                                     F_V[b], SEM[b][1])

    def fout_desc(k, b):
        return pltpu.make_async_copy(F_V[b], inst_out.at[pl.ds(base_of(k), CH)],
                                     SEM[b][2])

    def out_descs(k, b):
        base = base_of(k)
        return [
            pltpu.make_async_copy(GO_V[b], pool_out.at[pl.ds(base, CH)],
                                  SEM[b][3]),
            pltpu.make_async_copy(R_V[b], reu_out.at[pl.ds(base, CH)],
                                  SEM[b][4]),
            pltpu.make_async_copy(U_V[b], unc_out.at[pl.ds(base, CH)],
                                  SEM[b][5]),
            pltpu.make_async_copy(T_V[b], tag_out.at[pl.ds(base, CH)],
                                  SEM[b][6]),
        ]

    def bf16_round(v):
        u = plsc.bitcast(v, jnp.int32)
        r = u + (jnp.int32(0x7FFF) + ((u >> 16) & 1))
        return plsc.bitcast(r & jnp.int32(-65536), jnp.float32)

    def col(c):
        return jnp.full((L,), c, jnp.int32)

    def compute(b):
        @plsc.parallel_loop(0, NG, unroll=2)
        def mask_body(j):
            rows = j * L + iota
            x = plsc.load_gather(G_V[b], [rows, col(0)])
            y = plsc.load_gather(G_V[b], [rows, col(1)])
            z = plsc.load_gather(G_V[b], [rows, col(2)])
            xb = bf16_round(x)
            yb = bf16_round(y)
            zb = bf16_round(z)
            cx = xb * w00 + yb * w01 + zb * w02 + w03
            cy = xb * w10 + yb * w11 + zb * w12 + w13
            cz_ = xb * w20 + yb * w21 + zb * w22 + w23
            mask1 = cz_ > 1e-6
            cz = jnp.maximum(cz_, 1e-6)
            pxf = fx * (cx / cz) + cx0
            pyf = fy * (cy / cz) + cy0
            mask2 = ((pxf >= 0.0) & (pxf < 640.0)
                     & (pyf >= 0.0) & (pyf < 480.0))
            in_vox = ((x > n0) & (x < f0b) & (y > n1) & (y < f1b)
                      & (z > n2) & (z < f2b))
            mask_det = mask1 & mask2 & in_vox
            g23 = plsc.load_gather(G_V[b], [rows, col(23)])
            g24 = plsc.load_gather(G_V[b], [rows, col(24)])
            one = jnp.full((L,), 1.0, jnp.float32)
            half = jnp.full((L,), 0.5, jnp.float32)
            zero = jnp.zeros((L,), jnp.float32)
            plsc.store_scatter(G_V[b], [rows, col(24)],
                               jnp.where(in_vox, one, g24))
            T_V[b][pl.ds(j * L, L)] = jnp.where(
                mask_det, jnp.where(g23 == 1.0, half, zero), one)
            fm_v[pl.ds(j * L, L)] = jnp.where(in_vox, one, zero)
            um_v[pl.ds(j * L, L)] = jnp.where(mask_det, zero, one)

        @plsc.parallel_loop(0, GG, unroll=4)
        def g_body(kk):
            off = kk * L
            rowv = rtv[pl.ds(off, L)]
            colv = ctv[pl.ds(off, L)]
            fmv = plsc.load_gather(fm_v, [rowv])
            umv = plsc.load_gather(um_v, [rowv])
            gv = plsc.load_gather(G_V[b], [rowv, colv])
            plsc.store_scatter(GO_V[b], [rowv, colv], gv)
            plsc.store_scatter(R_V[b], [rowv, colv], gv * fmv)
            plsc.store_scatter(U_V[b], [rowv, colv], gv * umv)

        zero = jnp.zeros((L,), jnp.float32)

        def z_body(j, _):
            fmg = fm_v[pl.ds(j * L, L)]
            zm = fmg == 0.0

            @pl.when(jnp.any(zm))
            def _():
                for lane in range(L):
                    @pl.when(fmg[lane] == 0.0)
                    def _(lane=lane):
                        rowb = col(j * L + lane)
                        for cg in range(D // L):
                            plsc.store_scatter(F_V[b], [rowb, cg * L + iota],
                                               zero)

            return 0

        lax.fori_loop(0, NG, z_body, 0)

    gin_desc(0, 0).start()
    fin_desc(0, 0).start()

    def body(i, _):
        for bslot in (0, 1):
            k = i * 2 + bslot

            @pl.when(k < n_my)
            def _(k=k, bslot=bslot):
                gin_desc(k, bslot).wait()
                fin_desc(k, bslot).wait()

                @pl.when(k >= 2)
                def _():
                    for dsc in out_descs(k - 2, bslot):
                        dsc.wait()

                @pl.when(k + 1 < n_my)
                def _():
                    gin_desc(k + 1, 1 - bslot).start()

                compute(bslot)

                @pl.when(k >= 1)
                def _():
                    fout_desc(k - 1, 1 - bslot).wait()

                @pl.when(k + 1 < n_my)
                def _():
                    fin_desc(k + 1, 1 - bslot).start()

                fout_desc(k, bslot).start()
                for dsc in out_descs(k, bslot):
                    dsc.start()

        return 0

    lax.fori_loop(0, (n_my + 1) // 2, body, 0)

    for bslot in (0, 1):
        for k_last, with_fout in ((n_my - 2, False), (n_my - 1, True)):
            @pl.when((k_last >= 0) & (k_last % 2 == bslot))
            def _(k_last=k_last, bslot=bslot, with_fout=with_fout):
                if with_fout:
                    fout_desc(k_last, bslot).wait()
                for dsc in out_descs(k_last, bslot):
                    dsc.wait()


def kernel(gaussian_pool, instance_feature_pool, world2cam, cam_k,
           vox_origin, scene_size, mlvl_img_feat, anchor):
    M = gaussian_pool.shape[0]
    batch_size = mlvl_img_feat.shape[0]
    eps = jnp.float32(0.001)
    near = vox_origin + eps
    far = vox_origin + scene_size - eps
    w_rounded = world2cam[:3, :].astype(jnp.bfloat16).astype(jnp.float32)
    params = jnp.concatenate([
        w_rounded.reshape(-1),
        jnp.stack([cam_k[0, 0], cam_k[1, 1], cam_k[0, 2], cam_k[1, 2]]),
        near, far,
    ]).astype(jnp.float32)
    params_b = jnp.repeat(params, L)

    rowtab = jnp.repeat(jnp.arange(CH, dtype=jnp.int32), G)
    coltab = jnp.tile(jnp.arange(G, dtype=jnp.int32), CH)

    mesh = plsc.VectorSubcoreMesh(core_axis_name="c", subcore_axis_name="s")
    f32 = jnp.float32
    i32 = jnp.int32
    dma_sems = [pltpu.SemaphoreType.DMA] * 14
    call = pl.kernel(
        _sc_body,
        out_type=[
            jax.ShapeDtypeStruct((M, G), f32),
            jax.ShapeDtypeStruct((M,), f32),
            jax.ShapeDtypeStruct((M, G), f32),
            jax.ShapeDtypeStruct((M, G), f32),
            jax.ShapeDtypeStruct((M, D), f32),
        ],
        mesh=mesh,
        compiler_params=pltpu.CompilerParams(needs_layout_passes=False),
        scratch_types=[
            pltpu.VMEM((22 * L,), f32),
            pltpu.VMEM((CH * G,), i32),
            pltpu.VMEM((CH * G,), i32),
            pltpu.VMEM((CH,), f32),
            pltpu.VMEM((CH,), f32),
            pltpu.VMEM((CH, G), f32), pltpu.VMEM((CH, G), f32),
            pltpu.VMEM((CH, G), f32), pltpu.VMEM((CH, G), f32),
            pltpu.VMEM((CH, G), f32), pltpu.VMEM((CH, G), f32),
            pltpu.VMEM((CH, G), f32), pltpu.VMEM((CH, G), f32),
            pltpu.VMEM((CH, D), f32), pltpu.VMEM((CH, D), f32),
            pltpu.VMEM((CH,), f32), pltpu.VMEM((CH,), f32),
        ] + dma_sems,
    )
    pool_u, tag, reu, unc, inst = call(gaussian_pool, instance_feature_pool,
                                       params_b, rowtab, coltab)

    anchor_tiled = jnp.tile(anchor[None], (batch_size, 1, 1))
    return (pool_u, tag, reu, unc, inst, anchor_tiled)
